# baseline (speedup 1.0000x reference)
import jax
import jax.numpy as jnp
from jax.experimental import pallas as pl
from jax.experimental.pallas import tpu as pltpu


_CHUNK_ROWS = (5000, 5000)
_CHUNKS = len(_CHUNK_ROWS)


def _dma_tile4_wide_kernel(x_hbm, o_hbm, vbuf, in_sems, dup_sems, out_sems):
    n, d2 = vbuf.shape
    d = d2 // 2
    in_cps, dup_cps = [], []
    base = 0
    for c, h in enumerate(_CHUNK_ROWS):
        rows = pl.ds(base, h)
        base += h
        cp = pltpu.make_async_copy(
            x_hbm.at[rows, :], vbuf.at[rows, pl.ds(0, d)], in_sems.at[c])
        cp.start()
        in_cps.append(cp)
    base = 0
    for c, h in enumerate(_CHUNK_ROWS):
        in_cps[c].wait()
        rows = pl.ds(base, h)
        base += h
        cp = pltpu.make_async_copy(
            vbuf.at[rows, pl.ds(0, d)], vbuf.at[rows, pl.ds(d, d)],
            dup_sems.at[c])
        cp.start()
        dup_cps.append(cp)
    out_cps = []
    base = 0
    for c, h in enumerate(_CHUNK_ROWS):
        dup_cps[c].wait()
        rows = pl.ds(base, h)
        base += h
        for j in range(2):
            cp = pltpu.make_async_copy(
                vbuf.at[rows, :], o_hbm.at[rows, pl.ds(j * d2, d2)],
                out_sems.at[c, j])
            cp.start()
            out_cps.append(cp)
    for cp in out_cps:
        cp.wait()


def kernel(x, edge_index):
    del edge_index
    n, d = x.shape
    out = pl.pallas_call(
        _dma_tile4_wide_kernel,
        in_specs=[pl.BlockSpec(memory_space=pl.ANY)],
        out_specs=pl.BlockSpec(memory_space=pl.ANY),
        out_shape=jax.ShapeDtypeStruct((n, 4 * d), x.dtype),
        scratch_shapes=[
            pltpu.VMEM((n, 2 * d), x.dtype),
            pltpu.SemaphoreType.DMA((_CHUNKS,)),
            pltpu.SemaphoreType.DMA((_CHUNKS,)),
            pltpu.SemaphoreType.DMA((_CHUNKS, 2)),
        ],
    )(x)
    return out
